# baseline (speedup 1.0000x reference)
import functools

import jax
import jax.numpy as jnp
from jax import lax
from jax.experimental import pallas as pl
from jax.experimental.pallas import tpu as pltpu
from jax.experimental.pallas import tpu_sc as plsc

_B, _V, _F = 64, 512, 128
_NS, _NLR, _K = 4, 32, 16
_NSLR = _NS + _NLR
_NAGG = 2 * _NLR


def _phase_a_body(x_ref, wx_ref, wm_ref, b_ref, s_ref, lr_ref):
    xb = x_ref[0]
    xm = jnp.mean(xb, axis=0, keepdims=True)
    y = jnp.dot(xb, wx_ref[...], preferred_element_type=jnp.float32)
    y = y + jnp.dot(xm, wm_ref[...], preferred_element_type=jnp.float32)
    y = jnp.maximum(y + b_ref[...], 0.0)
    s_ref[0] = y[:, :_NS]
    lr_ref[0] = y[:, _NS:]


def _phase_a(x, wx, wm, b):
    return pl.pallas_call(
        _phase_a_body,
        grid=(_B,),
        in_specs=[
            pl.BlockSpec((1, _V, _F), lambda i: (i, 0, 0)),
            pl.BlockSpec((_F, _NSLR), lambda i: (0, 0)),
            pl.BlockSpec((_F, _NSLR), lambda i: (0, 0)),
            pl.BlockSpec((1, _NSLR), lambda i: (0, 0)),
        ],
        out_specs=[
            pl.BlockSpec((1, _V, _NS), lambda i: (i, 0, 0)),
            pl.BlockSpec((1, _V, _NLR), lambda i: (i, 0, 0)),
        ],
        out_shape=[
            jax.ShapeDtypeStruct((_B, _V, _NS), jnp.float32),
            jax.ShapeDtypeStruct((_B, _V, _NLR), jnp.float32),
        ],
    )(x, wx, wm, b)


def _phase_b_body(s_hbm, lr_hbm, agg_hbm,
                  s_v, lr_v, out_v, dbuf_v,
                  bltj_v, bltd_v, beqj_v):
    info = plsc.get_sparse_core_info()
    nc = info.num_cores
    wid = lax.axis_index("s") * nc + lax.axis_index("c")
    iota = jnp.arange(16, dtype=jnp.int32)
    inf16 = jnp.full((16,), jnp.inf, jnp.float32)
    zero16i = jnp.zeros((16,), jnp.int32)

    def one_batch(b):
        pltpu.sync_copy(s_hbm.at[b], s_v.at[pl.ds(0, _V * _NS)])
        pltpu.sync_copy(lr_hbm.at[b], lr_v.at[pl.ds(0, _V * _NLR)])

        def row(i, _):
            srow = s_v[pl.ds(i * _NS, 16)]
            si0 = srow[0]
            si1 = srow[1]
            si2 = srow[2]
            si3 = srow[3]

            def chunk(c, carry):
                jv = iota + c * 16
                jv4 = jv * _NS
                g0 = plsc.load_gather(s_v, [jv4])
                g1 = plsc.load_gather(s_v, [jv4 + 1])
                g2 = plsc.load_gather(s_v, [jv4 + 2])
                g3 = plsc.load_gather(s_v, [jv4 + 3])
                e0 = g0 - si0
                e1 = g1 - si1
                e2 = g2 - si2
                e3 = g3 - si3
                d = e0 * e0 + e1 * e1 + e2 * e2 + e3 * e3
                dbuf_v[pl.ds(c * 16, 16)] = d

                def merge(carry):
                    best_d, best_j, d17, _ = carry
                    ds, js = plsc.sort_key_val(d, jv)
                    rb = lax.rev(best_d, (0,))
                    rbj = lax.rev(best_j, (0,))
                    take = ds < rb
                    nd = jnp.where(take, ds, rb)
                    nj = jnp.where(take, js, rbj)
                    hi = jnp.where(take, rb, ds)
                    d17 = jnp.minimum(d17, hi)
                    best_d, best_j = plsc.sort_key_val(nd, nj)
                    return best_d, best_j, d17, jnp.max(best_d)

                cnt = plsc.all_reduce_population_count(d <= carry[3])
                return lax.cond(cnt[0] > 0, merge, lambda cc: cc, carry)

            best_d, best_j, d17v, t = lax.fori_loop(
                0, 32, chunk,
                (inf16, zero16i, inf16, jnp.float32(jnp.inf)))
            t17 = jnp.min(d17v)

            def fixup(_):
                c_lt = jnp.sum((best_d < t).astype(jnp.int32))

                def fchunk(c, offs):
                    d = dbuf_v[pl.ds(c * 16, 16)]
                    jv = iota + c * 16

                    def compact(offs):
                        off_lt, off_eq = offs
                        mlt = d < t
                        meq = d == t
                        plsc.store_compressed(bltj_v.at[pl.ds(off_lt, 16)],
                                              jv, mask=mlt)
                        plsc.store_compressed(bltd_v.at[pl.ds(off_lt, 16)],
                                              d, mask=mlt)
                        plsc.store_compressed(beqj_v.at[pl.ds(off_eq, 16)],
                                              jv, mask=meq)
                        off_lt = off_lt + jnp.sum(mlt.astype(jnp.int32))
                        off_eq = off_eq + jnp.sum(meq.astype(jnp.int32))
                        return off_lt, off_eq

                    cnt = plsc.all_reduce_population_count(d <= t)
                    return lax.cond(cnt[0] > 0, compact, lambda oo: oo, offs)

                lax.fori_loop(0, 32, fchunk,
                              (jnp.int32(0), jnp.int32(0)))
                mk = iota < c_lt
                ja = plsc.load_gather(bltj_v, [iota])
                da = plsc.load_gather(bltd_v, [iota])
                jb = plsc.load_gather(
                    beqj_v, [jnp.maximum(iota - c_lt, 0)])
                return (jnp.where(mk, ja, jb),
                        jnp.where(mk, da, t))

            j_sel, d_sel = lax.cond(
                t17 == t, fixup, lambda _: (best_j, best_d), None)
            w = jnp.exp(-10.0 * d_sel)

            z = jnp.zeros((16,), jnp.float32)
            m0, m1, x0, x1 = z, z, z, z
            for k in range(16):
                jk = j_sel[k]
                wk = w[k]
                r0 = wk * lr_v[pl.ds(jk * _NLR, 16)]
                r1 = wk * lr_v[pl.ds(jk * _NLR + 16, 16)]
                m0 = m0 + r0
                m1 = m1 + r1
                x0 = jnp.maximum(x0, r0)
                x1 = jnp.maximum(x1, r1)
            out_v[pl.ds(i * _NAGG, 16)] = m0
            out_v[pl.ds(i * _NAGG + 16, 16)] = m1
            out_v[pl.ds(i * _NAGG + 32, 16)] = x0
            out_v[pl.ds(i * _NAGG + 48, 16)] = x1
            return 0

        lax.fori_loop(0, _V, row, 0)
        pltpu.sync_copy(out_v, agg_hbm.at[b])

    for bi in range(_B // 32):
        one_batch(wid * (_B // 32) + bi)


def _phase_b(s, lr):
    mesh = plsc.VectorSubcoreMesh(core_axis_name="c", subcore_axis_name="s")
    f = pl.kernel(
        _phase_b_body,
        out_type=jax.ShapeDtypeStruct((_B, _V * _NAGG), jnp.float32),
        mesh=mesh,
        compiler_params=pltpu.CompilerParams(needs_layout_passes=False),
        scratch_types=[
            pltpu.VMEM((_V * _NS + 16,), jnp.float32),
            pltpu.VMEM((_V * _NLR + 16,), jnp.float32),
            pltpu.VMEM((_V * _NAGG,), jnp.float32),
            pltpu.VMEM((_V,), jnp.float32),
            pltpu.VMEM((32,), jnp.int32),
            pltpu.VMEM((32,), jnp.float32),
            pltpu.VMEM((_V + 16,), jnp.int32),
        ],
    )
    agg = f(s.reshape(_B, _V * _NS), lr.reshape(_B, _V * _NLR))
    return agg.reshape(_B, _V, _NAGG)


def _phase_c_body(x_ref, agg_ref, wa_ref, wb_ref, wc_ref, b_ref, o_ref):
    xb = x_ref[0]
    xm = jnp.mean(xb, axis=0, keepdims=True)
    acc = jnp.dot(xb, wa_ref[...], preferred_element_type=jnp.float32)
    acc = acc + jnp.dot(xm, wb_ref[...], preferred_element_type=jnp.float32)
    acc = acc + jnp.dot(agg_ref[0], wc_ref[...],
                        preferred_element_type=jnp.float32)
    o_ref[0] = jnp.maximum(acc + b_ref[...], 0.0)


def _phase_c(x, agg, wa, wb, wc, b):
    n_out = wa.shape[1]
    return pl.pallas_call(
        _phase_c_body,
        grid=(_B,),
        in_specs=[
            pl.BlockSpec((1, _V, _F), lambda i: (i, 0, 0)),
            pl.BlockSpec((1, _V, _NAGG), lambda i: (i, 0, 0)),
            pl.BlockSpec((_F, n_out), lambda i: (0, 0)),
            pl.BlockSpec((_F, n_out), lambda i: (0, 0)),
            pl.BlockSpec((_NAGG, n_out), lambda i: (0, 0)),
            pl.BlockSpec((1, n_out), lambda i: (0, 0)),
        ],
        out_specs=pl.BlockSpec((1, _V, n_out), lambda i: (i, 0, 0)),
        out_shape=jax.ShapeDtypeStruct((_B, _V, n_out), jnp.float32),
    )(x, agg, wa, wb, wc, b)


@jax.jit
def kernel(x, W_slr, b_slr, W_out, b_out):
    f = _F
    s, lr = _phase_a(x, W_slr[:f], W_slr[f:], b_slr.reshape(1, -1))
    agg = _phase_b(s, lr)
    wc = jnp.concatenate(
        [W_out[2 * f:2 * f + _NLR] / float(_K), W_out[2 * f + _NLR:]], axis=0)
    out = _phase_c(x, agg, W_out[:f], W_out[f:2 * f], wc,
                   b_out.reshape(1, -1))
    return out


# speedup vs baseline: 2.5393x; 2.5393x over previous
import functools

import jax
import jax.numpy as jnp
from jax import lax
from jax.experimental import pallas as pl
from jax.experimental.pallas import tpu as pltpu
from jax.experimental.pallas import tpu_sc as plsc

_B, _V, _F = 64, 512, 128
_NS, _NLR, _K = 4, 32, 16
_NSLR = _NS + _NLR
_NAGG = 2 * _NLR


def _phase_a_body(x_ref, wx_ref, wm_ref, b_ref, s_ref, lr_ref):
    xb = x_ref[0]
    xm = jnp.mean(xb, axis=0, keepdims=True)
    y = jnp.dot(xb, wx_ref[...], preferred_element_type=jnp.float32)
    y = y + jnp.dot(xm, wm_ref[...], preferred_element_type=jnp.float32)
    y = jnp.maximum(y + b_ref[...], 0.0)
    s_ref[0] = y[:, :_NS]
    lr_ref[0] = y[:, _NS:]


def _phase_a(x, wx, wm, b):
    return pl.pallas_call(
        _phase_a_body,
        grid=(_B,),
        in_specs=[
            pl.BlockSpec((1, _V, _F), lambda i: (i, 0, 0)),
            pl.BlockSpec((_F, _NSLR), lambda i: (0, 0)),
            pl.BlockSpec((_F, _NSLR), lambda i: (0, 0)),
            pl.BlockSpec((1, _NSLR), lambda i: (0, 0)),
        ],
        out_specs=[
            pl.BlockSpec((1, _V, _NS), lambda i: (i, 0, 0)),
            pl.BlockSpec((1, _V, _NLR), lambda i: (i, 0, 0)),
        ],
        out_shape=[
            jax.ShapeDtypeStruct((_B, _V, _NS), jnp.float32),
            jax.ShapeDtypeStruct((_B, _V, _NLR), jnp.float32),
        ],
    )(x, wx, wm, b)


def _phase_b_body(s_hbm, lr_hbm, agg_hbm,
                  s_v, lr_v, out_v, dbuf_v,
                  bltj_v, bltd_v, beqj_v):
    info = plsc.get_sparse_core_info()
    nc = info.num_cores
    wid = lax.axis_index("s") * nc + lax.axis_index("c")
    iota = jnp.arange(16, dtype=jnp.int32)
    inf16 = jnp.full((16,), jnp.inf, jnp.float32)
    zero16i = jnp.zeros((16,), jnp.int32)

    def one_batch(b):
        pltpu.sync_copy(s_hbm.at[b], s_v.at[pl.ds(0, _V * _NS)])
        pltpu.sync_copy(lr_hbm.at[b], lr_v.at[pl.ds(0, _V * _NLR)])

        def row(i, _):
            srow = s_v[pl.ds(i * _NS, 16)]
            si0 = srow[0]
            si1 = srow[1]
            si2 = srow[2]
            si3 = srow[3]

            def chunk(c, carry):
                jv = iota + c * 16
                jv4 = jv * _NS
                g0 = plsc.load_gather(s_v, [jv4])
                g1 = plsc.load_gather(s_v, [jv4 + 1])
                g2 = plsc.load_gather(s_v, [jv4 + 2])
                g3 = plsc.load_gather(s_v, [jv4 + 3])
                e0 = g0 - si0
                e1 = g1 - si1
                e2 = g2 - si2
                e3 = g3 - si3
                d = e0 * e0 + e1 * e1 + e2 * e2 + e3 * e3
                dbuf_v[pl.ds(c * 16, 16)] = d
                best_d, best_j, d17 = carry
                ds, js = plsc.sort_key_val(d, jv)
                rb = lax.rev(best_d, (0,))
                rbj = lax.rev(best_j, (0,))
                take = ds < rb
                nd = jnp.where(take, ds, rb)
                nj = jnp.where(take, js, rbj)
                hi = jnp.where(take, rb, ds)
                d17 = jnp.minimum(d17, hi)
                best_d, best_j = plsc.sort_key_val(nd, nj)
                return best_d, best_j, d17

            best_d, best_j, d17v = lax.fori_loop(
                0, 32, chunk, (inf16, zero16i, inf16))
            t = jnp.max(best_d)
            t17 = jnp.min(d17v)

            def fixup(_):
                c_lt = jnp.sum((best_d < t).astype(jnp.int32))

                def fchunk(c, offs):
                    d = dbuf_v[pl.ds(c * 16, 16)]
                    jv = iota + c * 16

                    def compact(offs):
                        off_lt, off_eq = offs
                        mlt = d < t
                        meq = d == t
                        plsc.store_compressed(bltj_v.at[pl.ds(off_lt, 16)],
                                              jv, mask=mlt)
                        plsc.store_compressed(bltd_v.at[pl.ds(off_lt, 16)],
                                              d, mask=mlt)
                        plsc.store_compressed(beqj_v.at[pl.ds(off_eq, 16)],
                                              jv, mask=meq)
                        off_lt = off_lt + jnp.sum(mlt.astype(jnp.int32))
                        off_eq = off_eq + jnp.sum(meq.astype(jnp.int32))
                        return off_lt, off_eq

                    cnt = plsc.all_reduce_population_count(d <= t)
                    return lax.cond(cnt[0] > 0, compact, lambda oo: oo, offs)

                lax.fori_loop(0, 32, fchunk,
                              (jnp.int32(0), jnp.int32(0)))
                mk = iota < c_lt
                ja = plsc.load_gather(bltj_v, [iota])
                da = plsc.load_gather(bltd_v, [iota])
                jb = plsc.load_gather(
                    beqj_v, [jnp.maximum(iota - c_lt, 0)])
                return (jnp.where(mk, ja, jb),
                        jnp.where(mk, da, t))

            j_sel, d_sel = lax.cond(
                t17 == t, fixup, lambda _: (best_j, best_d), None)
            w = jnp.exp(-10.0 * d_sel)

            z = jnp.zeros((16,), jnp.float32)
            m0, m1, x0, x1 = z, z, z, z
            for k in range(16):
                jk = j_sel[k]
                wk = w[k]
                r0 = wk * lr_v[pl.ds(jk * _NLR, 16)]
                r1 = wk * lr_v[pl.ds(jk * _NLR + 16, 16)]
                m0 = m0 + r0
                m1 = m1 + r1
                x0 = jnp.maximum(x0, r0)
                x1 = jnp.maximum(x1, r1)
            out_v[pl.ds(i * _NAGG, 16)] = m0
            out_v[pl.ds(i * _NAGG + 16, 16)] = m1
            out_v[pl.ds(i * _NAGG + 32, 16)] = x0
            out_v[pl.ds(i * _NAGG + 48, 16)] = x1
            return 0

        lax.fori_loop(0, _V, row, 0)
        pltpu.sync_copy(out_v, agg_hbm.at[b])

    for bi in range(_B // 32):
        one_batch(wid * (_B // 32) + bi)


def _phase_b(s, lr):
    mesh = plsc.VectorSubcoreMesh(core_axis_name="c", subcore_axis_name="s")
    f = pl.kernel(
        _phase_b_body,
        out_type=jax.ShapeDtypeStruct((_B, _V * _NAGG), jnp.float32),
        mesh=mesh,
        compiler_params=pltpu.CompilerParams(needs_layout_passes=False),
        scratch_types=[
            pltpu.VMEM((_V * _NS + 16,), jnp.float32),
            pltpu.VMEM((_V * _NLR + 16,), jnp.float32),
            pltpu.VMEM((_V * _NAGG,), jnp.float32),
            pltpu.VMEM((_V,), jnp.float32),
            pltpu.VMEM((32,), jnp.int32),
            pltpu.VMEM((32,), jnp.float32),
            pltpu.VMEM((_V + 16,), jnp.int32),
        ],
    )
    agg = f(s.reshape(_B, _V * _NS), lr.reshape(_B, _V * _NLR))
    return agg.reshape(_B, _V, _NAGG)


def _phase_c_body(x_ref, agg_ref, wa_ref, wb_ref, wc_ref, b_ref, o_ref):
    xb = x_ref[0]
    xm = jnp.mean(xb, axis=0, keepdims=True)
    acc = jnp.dot(xb, wa_ref[...], preferred_element_type=jnp.float32)
    acc = acc + jnp.dot(xm, wb_ref[...], preferred_element_type=jnp.float32)
    acc = acc + jnp.dot(agg_ref[0], wc_ref[...],
                        preferred_element_type=jnp.float32)
    o_ref[0] = jnp.maximum(acc + b_ref[...], 0.0)


def _phase_c(x, agg, wa, wb, wc, b):
    n_out = wa.shape[1]
    return pl.pallas_call(
        _phase_c_body,
        grid=(_B,),
        in_specs=[
            pl.BlockSpec((1, _V, _F), lambda i: (i, 0, 0)),
            pl.BlockSpec((1, _V, _NAGG), lambda i: (i, 0, 0)),
            pl.BlockSpec((_F, n_out), lambda i: (0, 0)),
            pl.BlockSpec((_F, n_out), lambda i: (0, 0)),
            pl.BlockSpec((_NAGG, n_out), lambda i: (0, 0)),
            pl.BlockSpec((1, n_out), lambda i: (0, 0)),
        ],
        out_specs=pl.BlockSpec((1, _V, n_out), lambda i: (i, 0, 0)),
        out_shape=jax.ShapeDtypeStruct((_B, _V, n_out), jnp.float32),
    )(x, agg, wa, wb, wc, b)


@jax.jit
def kernel(x, W_slr, b_slr, W_out, b_out):
    f = _F
    s, lr = _phase_a(x, W_slr[:f], W_slr[f:], b_slr.reshape(1, -1))
    agg = _phase_b(s, lr)
    wc = jnp.concatenate(
        [W_out[2 * f:2 * f + _NLR] / float(_K), W_out[2 * f + _NLR:]], axis=0)
    out = _phase_c(x, agg, W_out[:f], W_out[f:2 * f], wc,
                   b_out.reshape(1, -1))
    return out


# speedup vs baseline: 3.4809x; 1.3708x over previous
import functools

import jax
import jax.numpy as jnp
from jax import lax
from jax.experimental import pallas as pl
from jax.experimental.pallas import tpu as pltpu
from jax.experimental.pallas import tpu_sc as plsc

_B, _V, _F = 64, 512, 128
_NS, _NLR, _K = 4, 32, 16
_NSLR = _NS + _NLR
_NAGG = 2 * _NLR


def _phase_a_body(x_ref, wx_ref, wm_ref, b_ref, s_ref, lr_ref):
    xb = x_ref[0]
    xm = jnp.mean(xb, axis=0, keepdims=True)
    y = jnp.dot(xb, wx_ref[...], preferred_element_type=jnp.float32)
    y = y + jnp.dot(xm, wm_ref[...], preferred_element_type=jnp.float32)
    y = jnp.maximum(y + b_ref[...], 0.0)
    s_ref[0] = y[:, :_NS]
    lr_ref[0] = y[:, _NS:]


def _phase_a(x, wx, wm, b):
    return pl.pallas_call(
        _phase_a_body,
        grid=(_B,),
        in_specs=[
            pl.BlockSpec((1, _V, _F), lambda i: (i, 0, 0)),
            pl.BlockSpec((_F, _NSLR), lambda i: (0, 0)),
            pl.BlockSpec((_F, _NSLR), lambda i: (0, 0)),
            pl.BlockSpec((1, _NSLR), lambda i: (0, 0)),
        ],
        out_specs=[
            pl.BlockSpec((1, _V, _NS), lambda i: (i, 0, 0)),
            pl.BlockSpec((1, _V, _NLR), lambda i: (i, 0, 0)),
        ],
        out_shape=[
            jax.ShapeDtypeStruct((_B, _V, _NS), jnp.float32),
            jax.ShapeDtypeStruct((_B, _V, _NLR), jnp.float32),
        ],
    )(x, wx, wm, b)


def _phase_b_body(s_hbm, lr_hbm, agg_hbm,
                  s_v, lr_v, out_v, dbuf_v,
                  bltj_v, bltd_v, beqj_v):
    info = plsc.get_sparse_core_info()
    nc = info.num_cores
    wid = lax.axis_index("s") * nc + lax.axis_index("c")
    iota = jnp.arange(16, dtype=jnp.int32)
    inf16 = jnp.full((16,), jnp.inf, jnp.float32)
    zero16i = jnp.zeros((16,), jnp.int32)

    def one_batch(b):
        pltpu.sync_copy(s_hbm.at[b], s_v.at[pl.ds(0, _V * _NS)])
        pltpu.sync_copy(lr_hbm.at[b], lr_v.at[pl.ds(0, _V * _NLR)])

        def finish_row(i, best_d, best_j, d17v, dbase):
            t = jnp.max(best_d)
            t17 = jnp.min(d17v)

            def fixup(_):
                c_lt = jnp.sum((best_d < t).astype(jnp.int32))

                def fchunk(c, offs):
                    d = dbuf_v[pl.ds(dbase + c * 16, 16)]
                    jv = iota + c * 16

                    def compact(offs):
                        off_lt, off_eq = offs
                        mlt = d < t
                        meq = d == t
                        plsc.store_compressed(bltj_v.at[pl.ds(off_lt, 16)],
                                              jv, mask=mlt)
                        plsc.store_compressed(bltd_v.at[pl.ds(off_lt, 16)],
                                              d, mask=mlt)
                        plsc.store_compressed(beqj_v.at[pl.ds(off_eq, 16)],
                                              jv, mask=meq)
                        off_lt = off_lt + jnp.sum(mlt.astype(jnp.int32))
                        off_eq = off_eq + jnp.sum(meq.astype(jnp.int32))
                        return off_lt, off_eq

                    cnt = plsc.all_reduce_population_count(d <= t)
                    return lax.cond(cnt[0] > 0, compact, lambda oo: oo, offs)

                lax.fori_loop(0, 32, fchunk,
                              (jnp.int32(0), jnp.int32(0)))
                mk = iota < c_lt
                ja = plsc.load_gather(bltj_v, [iota])
                da = plsc.load_gather(bltd_v, [iota])
                jb = plsc.load_gather(
                    beqj_v, [jnp.maximum(iota - c_lt, 0)])
                return (jnp.where(mk, ja, jb),
                        jnp.where(mk, da, t))

            j_sel, d_sel = lax.cond(
                t17 == t, fixup, lambda _: (best_j, best_d), None)
            w = jnp.exp(-10.0 * d_sel)

            z = jnp.zeros((16,), jnp.float32)
            m0, m1, x0, x1 = z, z, z, z
            for k in range(16):
                jk = j_sel[k]
                wk = w[k]
                r0 = wk * lr_v[pl.ds(jk * _NLR, 16)]
                r1 = wk * lr_v[pl.ds(jk * _NLR + 16, 16)]
                m0 = m0 + r0
                m1 = m1 + r1
                x0 = jnp.maximum(x0, r0)
                x1 = jnp.maximum(x1, r1)
            out_v[pl.ds(i * _NAGG, 16)] = m0
            out_v[pl.ds(i * _NAGG + 16, 16)] = m1
            out_v[pl.ds(i * _NAGG + 32, 16)] = x0
            out_v[pl.ds(i * _NAGG + 48, 16)] = x1

        def row_pair(p, _):
            ia = p * 2
            ib = ia + 1
            srow = s_v[pl.ds(ia * _NS, 16)]
            a0 = srow[0]
            a1 = srow[1]
            a2 = srow[2]
            a3 = srow[3]
            b0 = srow[4]
            b1 = srow[5]
            b2 = srow[6]
            b3 = srow[7]

            def chunk(c, carry):
                bdA, bjA, d17A, bdB, bjB, d17B = carry
                jv = iota + c * 16
                jv4 = jv * _NS
                g0 = plsc.load_gather(s_v, [jv4])
                g1 = plsc.load_gather(s_v, [jv4 + 1])
                g2 = plsc.load_gather(s_v, [jv4 + 2])
                g3 = plsc.load_gather(s_v, [jv4 + 3])
                ea0 = g0 - a0
                ea1 = g1 - a1
                ea2 = g2 - a2
                ea3 = g3 - a3
                dA = ea0 * ea0 + ea1 * ea1 + ea2 * ea2 + ea3 * ea3
                eb0 = g0 - b0
                eb1 = g1 - b1
                eb2 = g2 - b2
                eb3 = g3 - b3
                dB = eb0 * eb0 + eb1 * eb1 + eb2 * eb2 + eb3 * eb3
                dbuf_v[pl.ds(c * 16, 16)] = dA
                dbuf_v[pl.ds(_V + c * 16, 16)] = dB
                dsA, jsA = plsc.sort_key_val(dA, jv)
                dsB, jsB = plsc.sort_key_val(dB, jv)
                rbA = lax.rev(bdA, (0,))
                rbjA = lax.rev(bjA, (0,))
                takeA = dsA < rbA
                ndA = jnp.where(takeA, dsA, rbA)
                njA = jnp.where(takeA, jsA, rbjA)
                d17A = jnp.minimum(d17A, jnp.where(takeA, rbA, dsA))
                rbB = lax.rev(bdB, (0,))
                rbjB = lax.rev(bjB, (0,))
                takeB = dsB < rbB
                ndB = jnp.where(takeB, dsB, rbB)
                njB = jnp.where(takeB, jsB, rbjB)
                d17B = jnp.minimum(d17B, jnp.where(takeB, rbB, dsB))
                bdA, bjA = plsc.sort_key_val(ndA, njA)
                bdB, bjB = plsc.sort_key_val(ndB, njB)
                return bdA, bjA, d17A, bdB, bjB, d17B

            bdA, bjA, d17A, bdB, bjB, d17B = lax.fori_loop(
                0, 32, chunk,
                (inf16, zero16i, inf16, inf16, zero16i, inf16))
            finish_row(ia, bdA, bjA, d17A, 0)
            finish_row(ib, bdB, bjB, d17B, _V)
            return 0

        lax.fori_loop(0, _V // 2, row_pair, 0)
        pltpu.sync_copy(out_v, agg_hbm.at[b])

    for bi in range(_B // 32):
        one_batch(wid * (_B // 32) + bi)


def _phase_b(s, lr):
    mesh = plsc.VectorSubcoreMesh(core_axis_name="c", subcore_axis_name="s")
    f = pl.kernel(
        _phase_b_body,
        out_type=jax.ShapeDtypeStruct((_B, _V * _NAGG), jnp.float32),
        mesh=mesh,
        compiler_params=pltpu.CompilerParams(needs_layout_passes=False),
        scratch_types=[
            pltpu.VMEM((_V * _NS + 16,), jnp.float32),
            pltpu.VMEM((_V * _NLR + 16,), jnp.float32),
            pltpu.VMEM((_V * _NAGG,), jnp.float32),
            pltpu.VMEM((2 * _V,), jnp.float32),
            pltpu.VMEM((32,), jnp.int32),
            pltpu.VMEM((32,), jnp.float32),
            pltpu.VMEM((_V + 16,), jnp.int32),
        ],
    )
    agg = f(s.reshape(_B, _V * _NS), lr.reshape(_B, _V * _NLR))
    return agg.reshape(_B, _V, _NAGG)


def _phase_c_body(x_ref, agg_ref, wa_ref, wb_ref, wc_ref, b_ref, o_ref):
    xb = x_ref[0]
    xm = jnp.mean(xb, axis=0, keepdims=True)
    acc = jnp.dot(xb, wa_ref[...], preferred_element_type=jnp.float32)
    acc = acc + jnp.dot(xm, wb_ref[...], preferred_element_type=jnp.float32)
    acc = acc + jnp.dot(agg_ref[0], wc_ref[...],
                        preferred_element_type=jnp.float32)
    o_ref[0] = jnp.maximum(acc + b_ref[...], 0.0)


def _phase_c(x, agg, wa, wb, wc, b):
    n_out = wa.shape[1]
    return pl.pallas_call(
        _phase_c_body,
        grid=(_B,),
        in_specs=[
            pl.BlockSpec((1, _V, _F), lambda i: (i, 0, 0)),
            pl.BlockSpec((1, _V, _NAGG), lambda i: (i, 0, 0)),
            pl.BlockSpec((_F, n_out), lambda i: (0, 0)),
            pl.BlockSpec((_F, n_out), lambda i: (0, 0)),
            pl.BlockSpec((_NAGG, n_out), lambda i: (0, 0)),
            pl.BlockSpec((1, n_out), lambda i: (0, 0)),
        ],
        out_specs=pl.BlockSpec((1, _V, n_out), lambda i: (i, 0, 0)),
        out_shape=jax.ShapeDtypeStruct((_B, _V, n_out), jnp.float32),
    )(x, agg, wa, wb, wc, b)


@jax.jit
def kernel(x, W_slr, b_slr, W_out, b_out):
    f = _F
    s, lr = _phase_a(x, W_slr[:f], W_slr[f:], b_slr.reshape(1, -1))
    agg = _phase_b(s, lr)
    wc = jnp.concatenate(
        [W_out[2 * f:2 * f + _NLR] / float(_K), W_out[2 * f + _NLR:]], axis=0)
    out = _phase_c(x, agg, W_out[:f], W_out[f:2 * f], wc,
                   b_out.reshape(1, -1))
    return out


# speedup vs baseline: 3.8596x; 1.1088x over previous
import functools

import jax
import jax.numpy as jnp
from jax import lax
from jax.experimental import pallas as pl
from jax.experimental.pallas import tpu as pltpu
from jax.experimental.pallas import tpu_sc as plsc

_B, _V, _F = 64, 512, 128
_NS, _NLR, _K = 4, 32, 16
_NSLR = _NS + _NLR
_NAGG = 2 * _NLR


def _phase_a_body(x_ref, wx_ref, wm_ref, b_ref, s_ref, lr_ref):
    xb = x_ref[0]
    xm = jnp.mean(xb, axis=0, keepdims=True)
    y = jnp.dot(xb, wx_ref[...], preferred_element_type=jnp.float32)
    y = y + jnp.dot(xm, wm_ref[...], preferred_element_type=jnp.float32)
    y = jnp.maximum(y + b_ref[...], 0.0)
    s_ref[0] = y[:, :_NS]
    lr_ref[0] = y[:, _NS:]


def _phase_a(x, wx, wm, b):
    return pl.pallas_call(
        _phase_a_body,
        grid=(_B,),
        in_specs=[
            pl.BlockSpec((1, _V, _F), lambda i: (i, 0, 0)),
            pl.BlockSpec((_F, _NSLR), lambda i: (0, 0)),
            pl.BlockSpec((_F, _NSLR), lambda i: (0, 0)),
            pl.BlockSpec((1, _NSLR), lambda i: (0, 0)),
        ],
        out_specs=[
            pl.BlockSpec((1, _V, _NS), lambda i: (i, 0, 0)),
            pl.BlockSpec((1, _V, _NLR), lambda i: (i, 0, 0)),
        ],
        out_shape=[
            jax.ShapeDtypeStruct((_B, _V, _NS), jnp.float32),
            jax.ShapeDtypeStruct((_B, _V, _NLR), jnp.float32),
        ],
    )(x, wx, wm, b)


def _phase_b_body(s_hbm, lr_hbm, agg_hbm,
                  s_v, lr_v, out_v, dbuf_v,
                  bltj_v, bltd_v, beqj_v):
    info = plsc.get_sparse_core_info()
    nc = info.num_cores
    wid = lax.axis_index("s") * nc + lax.axis_index("c")
    iota = jnp.arange(16, dtype=jnp.int32)
    inf16 = jnp.full((16,), jnp.inf, jnp.float32)
    zero16i = jnp.zeros((16,), jnp.int32)

    def one_batch(b):
        pltpu.sync_copy(s_hbm.at[b], s_v.at[pl.ds(0, _V * _NS)])
        pltpu.sync_copy(lr_hbm.at[b], lr_v.at[pl.ds(0, _V * _NLR)])

        def finish_row(i, best_d, best_j, d17v, dbase):
            t = jnp.max(best_d)
            t17 = jnp.min(d17v)

            def fixup(_):
                c_lt = jnp.sum((best_d < t).astype(jnp.int32))

                def fchunk(c, offs):
                    d = dbuf_v[pl.ds(dbase + c * 16, 16)]
                    jv = iota + c * 16

                    def compact(offs):
                        off_lt, off_eq = offs
                        mlt = d < t
                        meq = d == t
                        plsc.store_compressed(bltj_v.at[pl.ds(off_lt, 16)],
                                              jv, mask=mlt)
                        plsc.store_compressed(bltd_v.at[pl.ds(off_lt, 16)],
                                              d, mask=mlt)
                        plsc.store_compressed(beqj_v.at[pl.ds(off_eq, 16)],
                                              jv, mask=meq)
                        off_lt = off_lt + jnp.sum(mlt.astype(jnp.int32))
                        off_eq = off_eq + jnp.sum(meq.astype(jnp.int32))
                        return off_lt, off_eq

                    cnt = plsc.all_reduce_population_count(d <= t)
                    return lax.cond(cnt[0] > 0, compact, lambda oo: oo, offs)

                lax.fori_loop(0, 32, fchunk,
                              (jnp.int32(0), jnp.int32(0)))
                mk = iota < c_lt
                ja = plsc.load_gather(bltj_v, [iota])
                da = plsc.load_gather(bltd_v, [iota])
                jb = plsc.load_gather(
                    beqj_v, [jnp.maximum(iota - c_lt, 0)])
                return (jnp.where(mk, ja, jb),
                        jnp.where(mk, da, t))

            j_sel, d_sel = lax.cond(
                t17 == t, fixup, lambda _: (best_j, best_d), None)
            w = jnp.exp(-10.0 * d_sel)

            z = jnp.zeros((16,), jnp.float32)
            m0, m1, x0, x1 = z, z, z, z
            for k in range(16):
                jk = j_sel[k]
                wk = w[k]
                r0 = wk * lr_v[pl.ds(jk * _NLR, 16)]
                r1 = wk * lr_v[pl.ds(jk * _NLR + 16, 16)]
                m0 = m0 + r0
                m1 = m1 + r1
                x0 = jnp.maximum(x0, r0)
                x1 = jnp.maximum(x1, r1)
            out_v[pl.ds(i * _NAGG, 16)] = m0
            out_v[pl.ds(i * _NAGG + 16, 16)] = m1
            out_v[pl.ds(i * _NAGG + 32, 16)] = x0
            out_v[pl.ds(i * _NAGG + 48, 16)] = x1

        n_rows = 4

        def row_group(p, _):
            i0 = p * n_rows
            srow = s_v[pl.ds(i0 * _NS, 16)]
            si = [[srow[r * _NS + c] for c in range(_NS)]
                  for r in range(n_rows)]

            def chunk(c, carry):
                jv = iota + c * 16
                jv4 = jv * _NS
                g = [plsc.load_gather(s_v, [jv4 + cc]) for cc in range(_NS)]
                nxt = []
                for r in range(n_rows):
                    bd, bj, d17 = carry[3 * r:3 * r + 3]
                    e = [g[cc] - si[r][cc] for cc in range(_NS)]
                    d = e[0] * e[0] + e[1] * e[1] + e[2] * e[2] + e[3] * e[3]
                    dbuf_v[pl.ds(r * _V + c * 16, 16)] = d
                    ds, js = plsc.sort_key_val(d, jv)
                    rb = lax.rev(bd, (0,))
                    rbj = lax.rev(bj, (0,))
                    take = ds < rb
                    nd = jnp.where(take, ds, rb)
                    nj = jnp.where(take, js, rbj)
                    d17 = jnp.minimum(d17, jnp.where(take, rb, ds))
                    bd, bj = plsc.sort_key_val(nd, nj)
                    nxt += [bd, bj, d17]
                return tuple(nxt)

            res = lax.fori_loop(
                0, 32, chunk, (inf16, zero16i, inf16) * n_rows)
            for r in range(n_rows):
                finish_row(i0 + r, res[3 * r], res[3 * r + 1],
                           res[3 * r + 2], r * _V)
            return 0

        lax.fori_loop(0, _V // n_rows, row_group, 0)
        pltpu.sync_copy(out_v, agg_hbm.at[b])

    for bi in range(_B // 32):
        one_batch(wid * (_B // 32) + bi)


def _phase_b(s, lr):
    mesh = plsc.VectorSubcoreMesh(core_axis_name="c", subcore_axis_name="s")
    f = pl.kernel(
        _phase_b_body,
        out_type=jax.ShapeDtypeStruct((_B, _V * _NAGG), jnp.float32),
        mesh=mesh,
        compiler_params=pltpu.CompilerParams(needs_layout_passes=False),
        scratch_types=[
            pltpu.VMEM((_V * _NS + 16,), jnp.float32),
            pltpu.VMEM((_V * _NLR + 16,), jnp.float32),
            pltpu.VMEM((_V * _NAGG,), jnp.float32),
            pltpu.VMEM((4 * _V,), jnp.float32),
            pltpu.VMEM((32,), jnp.int32),
            pltpu.VMEM((32,), jnp.float32),
            pltpu.VMEM((_V + 16,), jnp.int32),
        ],
    )
    agg = f(s.reshape(_B, _V * _NS), lr.reshape(_B, _V * _NLR))
    return agg.reshape(_B, _V, _NAGG)


def _phase_c_body(x_ref, agg_ref, wa_ref, wb_ref, wc_ref, b_ref, o_ref):
    xb = x_ref[0]
    xm = jnp.mean(xb, axis=0, keepdims=True)
    acc = jnp.dot(xb, wa_ref[...], preferred_element_type=jnp.float32)
    acc = acc + jnp.dot(xm, wb_ref[...], preferred_element_type=jnp.float32)
    acc = acc + jnp.dot(agg_ref[0], wc_ref[...],
                        preferred_element_type=jnp.float32)
    o_ref[0] = jnp.maximum(acc + b_ref[...], 0.0)


def _phase_c(x, agg, wa, wb, wc, b):
    n_out = wa.shape[1]
    return pl.pallas_call(
        _phase_c_body,
        grid=(_B,),
        in_specs=[
            pl.BlockSpec((1, _V, _F), lambda i: (i, 0, 0)),
            pl.BlockSpec((1, _V, _NAGG), lambda i: (i, 0, 0)),
            pl.BlockSpec((_F, n_out), lambda i: (0, 0)),
            pl.BlockSpec((_F, n_out), lambda i: (0, 0)),
            pl.BlockSpec((_NAGG, n_out), lambda i: (0, 0)),
            pl.BlockSpec((1, n_out), lambda i: (0, 0)),
        ],
        out_specs=pl.BlockSpec((1, _V, n_out), lambda i: (i, 0, 0)),
        out_shape=jax.ShapeDtypeStruct((_B, _V, n_out), jnp.float32),
    )(x, agg, wa, wb, wc, b)


@jax.jit
def kernel(x, W_slr, b_slr, W_out, b_out):
    f = _F
    s, lr = _phase_a(x, W_slr[:f], W_slr[f:], b_slr.reshape(1, -1))
    agg = _phase_b(s, lr)
    wc = jnp.concatenate(
        [W_out[2 * f:2 * f + _NLR] / float(_K), W_out[2 * f + _NLR:]], axis=0)
    out = _phase_c(x, agg, W_out[:f], W_out[f:2 * f], wc,
                   b_out.reshape(1, -1))
    return out
